# baseline (speedup 1.0000x reference)
import functools

import jax
import jax.numpy as jnp
from jax.experimental import pallas as pl
from jax.experimental.pallas import tpu as pltpu

L = 6
E = 16
TOPK = 2
D = 256
H = 8
HD = D // H
DFF = 1024
V = 32000
B = 1
T = 64
EPB = 8
S = E // EPB
VCHUNK = 6400
NH = V // VCHUNK


def _ln(x, g, b):
    mu = jnp.mean(x, axis=-1, keepdims=True)
    xc = x - mu
    var = jnp.mean(xc * xc, axis=-1, keepdims=True)
    return xc * jax.lax.rsqrt(var + 1e-5) * g + b


def _embed_kernel(tok_ref, wte_ref, wpe_ref, out_ref):
    out_ref[...] = wte_ref[...] + wpe_ref[...]


def _embed(tokens, wte, wpe):
    grid_spec = pltpu.PrefetchScalarGridSpec(
        num_scalar_prefetch=1,
        grid=(T,),
        in_specs=[
            pl.BlockSpec((1, 1, D), lambda i, tok: (tok[i], 0, 0)),
            pl.BlockSpec((1, 1, D), lambda i, tok: (i, 0, 0)),
        ],
        out_specs=pl.BlockSpec((1, 1, D), lambda i, tok: (i, 0, 0)),
    )
    out = pl.pallas_call(
        _embed_kernel,
        grid_spec=grid_spec,
        out_shape=jax.ShapeDtypeStruct((T, 1, D), jnp.float32),
    )(tokens, wte.reshape(V, 1, D), wpe.reshape(T, 1, D))
    return out.reshape(T, D)


def _dot_t(a, b):
    return jax.lax.dot_general(
        a.astype(jnp.bfloat16), b.astype(jnp.bfloat16),
        (((1,), (1,)), ((), ())),
        preferred_element_type=jnp.float32,
    )


def _layers_kernel(
    x0_ref, n0g_ref, n0b_ref, ln1g_ref, ln1b_ref, ln2g_ref, ln2b_ref,
    qkv_ref, proj_ref, rw_ref, fc1_ref, fc2_ref, hw_ref, out_ref,
    x_s, h_s, mp_s, delta_s,
):
    i = pl.program_id(0)
    e = jax.lax.rem(i, S)

    @pl.when(i == 0)
    def _init():
        x_s[...] = _ln(x0_ref[...], n0g_ref[0], n0b_ref[0])

    @pl.when((e == 0) & (i < L * S))
    def _attn_and_router():
        x = x_s[...]
        xn = _ln(x, ln1g_ref[0, 0], ln1b_ref[0, 0])
        qkvm = _dot_t(xn, qkv_ref[0])
        scale = 1.0 / (HD ** 0.5)
        row = jax.lax.broadcasted_iota(jnp.int32, (T, T), 0)
        col = jax.lax.broadcasted_iota(jnp.int32, (T, T), 1)
        causal = row >= col
        ys = []
        for h in range(H):
            q = qkvm[:, h * HD:(h + 1) * HD]
            k = qkvm[:, D + h * HD:D + (h + 1) * HD]
            v = qkvm[:, 2 * D + h * HD:2 * D + (h + 1) * HD]
            att = _dot_t(q, k) * scale
            att = jnp.where(causal, att, -1e9)
            att = jax.nn.softmax(att, axis=-1)
            ys.append(
                jax.lax.dot_general(
                    att.astype(jnp.bfloat16), v.astype(jnp.bfloat16),
                    (((1,), (0,)), ((), ())),
                    preferred_element_type=jnp.float32,
                )
            )
        y = jnp.concatenate(ys, axis=-1)
        x = x + _dot_t(y, proj_ref[0])
        x_s[...] = x
        h_ = _ln(x, ln2g_ref[0, 0], ln2b_ref[0, 0])
        h_s[...] = h_
        scores = _dot_t(h_, rw_ref[0])
        probs = jax.nn.softmax(scores, axis=-1)
        lane = jax.lax.broadcasted_iota(jnp.int32, (T, E), 1)
        big = jnp.int32(2 * E)
        m1 = jnp.max(scores, axis=-1, keepdims=True)
        j1 = jnp.min(jnp.where(scores == m1, lane, big), axis=-1, keepdims=True)
        oh1 = lane == j1
        s2 = jnp.where(oh1, -jnp.inf, scores)
        m2 = jnp.max(s2, axis=-1, keepdims=True)
        j2 = jnp.min(jnp.where(s2 == m2, lane, big), axis=-1, keepdims=True)
        mask = oh1 | (lane == j2)
        mp = jnp.where(mask, probs, 0.0)
        mp = mp / (jnp.sum(mp, axis=-1, keepdims=True) + 1e-8)
        mp_s[...] = mp
        delta_s[...] = jnp.zeros_like(delta_s)

    @pl.when(i < L * S)
    def _experts():
        h_ = h_s[...]
        lane = jax.lax.broadcasted_iota(jnp.int32, (T, E), 1)
        mp = mp_s[...]
        acc = delta_s[...]
        for j in range(EPB):
            a1 = jnp.maximum(_dot_t(h_, fc1_ref[0, j]), 0.0)
            eo = _dot_t(a1, fc2_ref[0, j])
            w = jnp.sum(
                jnp.where(lane == e * EPB + j, mp, 0.0), axis=-1,
                keepdims=True,
            )
            acc = acc + w * eo
        delta_s[...] = acc

        @pl.when(e == S - 1)
        def _finish_layer():
            x_s[...] += delta_s[...]

    @pl.when(i >= L * S)
    def _head():
        out_ref[...] = _dot_t(x_s[...], hw_ref[...])


def _fwd(x0, n0g, n0b, ln1g, ln1b, ln2g, ln2b, qkv_w, proj_w, rw, fc1, fc2,
         head_w):
    lS = L * S

    def lidx(i):
        return jnp.minimum(i // S, L - 1)

    def eidx(i):
        return jnp.where(i < lS, jax.lax.rem(i, S), S - 1)

    def hidx(i):
        return jnp.maximum(i - lS, 0)

    return pl.pallas_call(
        _layers_kernel,
        grid=(lS + NH,),
        in_specs=[
            pl.BlockSpec((T, D), lambda i: (0, 0)),
            pl.BlockSpec((1, D), lambda i: (0, 0)),
            pl.BlockSpec((1, D), lambda i: (0, 0)),
            pl.BlockSpec((1, 1, D), lambda i: (lidx(i), 0, 0)),
            pl.BlockSpec((1, 1, D), lambda i: (lidx(i), 0, 0)),
            pl.BlockSpec((1, 1, D), lambda i: (lidx(i), 0, 0)),
            pl.BlockSpec((1, 1, D), lambda i: (lidx(i), 0, 0)),
            pl.BlockSpec((1, 3 * D, D), lambda i: (lidx(i), 0, 0)),
            pl.BlockSpec((1, D, D), lambda i: (lidx(i), 0, 0)),
            pl.BlockSpec((1, E, D), lambda i: (lidx(i), 0, 0)),
            pl.BlockSpec((1, EPB, DFF, D),
                         lambda i: (lidx(i), eidx(i), 0, 0)),
            pl.BlockSpec((1, EPB, D, DFF),
                         lambda i: (lidx(i), eidx(i), 0, 0)),
            pl.BlockSpec((VCHUNK, D), lambda i: (hidx(i), 0)),
        ],
        out_specs=pl.BlockSpec((T, VCHUNK), lambda i: (0, hidx(i))),
        out_shape=jax.ShapeDtypeStruct((T, V), jnp.float32),
        scratch_shapes=[
            pltpu.VMEM((T, D), jnp.float32),
            pltpu.VMEM((T, D), jnp.float32),
            pltpu.VMEM((T, E), jnp.float32),
            pltpu.VMEM((T, D), jnp.float32),
        ],
        compiler_params=pltpu.CompilerParams(
            dimension_semantics=("arbitrary",),
        ),
    )(x0, n0g, n0b, ln1g, ln1b, ln2g, ln2b, qkv_w, proj_w, rw, fc1, fc2,
      head_w)


@jax.jit
def _run(tokens, wte, wpe, norm0_g, norm0_b, ln1_g, ln1_b, ln2_g, ln2_b,
         attn_qkv_w, attn_proj_w, router_w, expert_fc1, expert_fc2, lm_head_w):
    tok = tokens.reshape(T).astype(jnp.int32)
    x0 = _embed(tok, wte, wpe)
    logits = _fwd(
        x0,
        norm0_g.reshape(1, D), norm0_b.reshape(1, D),
        ln1_g.reshape(L, 1, D), ln1_b.reshape(L, 1, D),
        ln2_g.reshape(L, 1, D), ln2_b.reshape(L, 1, D),
        attn_qkv_w, attn_proj_w, router_w, expert_fc1, expert_fc2,
        lm_head_w,
    )
    return logits.reshape(B, T, V)


def kernel(tokens, wte, wpe, norm0_g, norm0_b, ln1_g, ln1_b, ln2_g, ln2_b,
           attn_qkv_w, attn_proj_w, router_w, expert_fc1, expert_fc2,
           lm_head_w):
    return _run(tokens, wte, wpe, norm0_g, norm0_b, ln1_g, ln1_b, ln2_g,
                ln2_b, attn_qkv_w, attn_proj_w, router_w, expert_fc1,
                expert_fc2, lm_head_w)


# speedup vs baseline: 1.0012x; 1.0012x over previous
import jax
import jax.numpy as jnp
from jax.experimental import pallas as pl
from jax.experimental.pallas import tpu as pltpu

L = 6
E = 16
TOPK = 2
D = 256
H = 8
HD = D // H
DFF = 1024
V = 32000
B = 1
T = 64
EPB = 8
S = E // EPB
VCHUNK = 6400
NH = V // VCHUNK


def _ln(x, g, b):
    mu = jnp.mean(x, axis=-1, keepdims=True)
    xc = x - mu
    var = jnp.mean(xc * xc, axis=-1, keepdims=True)
    return xc / jnp.sqrt(var + 1e-5) * g + b


def _embed_kernel(tok_ref, wte_ref, wpe_ref, out_ref):
    out_ref[...] = wte_ref[...] + wpe_ref[...]


def _embed(tokens, wte, wpe):
    grid_spec = pltpu.PrefetchScalarGridSpec(
        num_scalar_prefetch=1,
        grid=(T,),
        in_specs=[
            pl.BlockSpec((1, 1, D), lambda i, tok: (tok[i], 0, 0)),
            pl.BlockSpec((1, 1, D), lambda i, tok: (i, 0, 0)),
        ],
        out_specs=pl.BlockSpec((1, 1, D), lambda i, tok: (i, 0, 0)),
    )
    out = pl.pallas_call(
        _embed_kernel,
        grid_spec=grid_spec,
        out_shape=jax.ShapeDtypeStruct((T, 1, D), jnp.float32),
    )(tokens, wte.reshape(V, 1, D), wpe.reshape(T, 1, D))
    return out.reshape(T, D)


def _dot_t(a, b):
    return jax.lax.dot_general(
        a.astype(jnp.bfloat16), b.astype(jnp.bfloat16),
        (((1,), (1,)), ((), ())),
        preferred_element_type=jnp.float32,
    )


def _layers_kernel(
    x0_ref, n0g_ref, n0b_ref, ln1g_ref, ln1b_ref, ln2g_ref, ln2b_ref,
    qkv_ref, proj_ref, rw_ref, fc1_ref, fc2_ref, hw_ref, out_ref,
    x_s, h_s, mp_s, delta_s,
):
    i = pl.program_id(0)
    e = jax.lax.rem(i, S)

    @pl.when(i == 0)
    def _init():
        x_s[...] = _ln(x0_ref[...], n0g_ref[0], n0b_ref[0])

    @pl.when((e == 0) & (i < L * S))
    def _attn_and_router():
        x = x_s[...]
        xn = _ln(x, ln1g_ref[0, 0], ln1b_ref[0, 0])
        qkvm = _dot_t(xn, qkv_ref[0])
        scale = jnp.float32(HD ** 0.5)
        row = jax.lax.broadcasted_iota(jnp.int32, (T, T), 0)
        col = jax.lax.broadcasted_iota(jnp.int32, (T, T), 1)
        causal = row >= col
        ys = []
        for h in range(H):
            q = qkvm[:, h * HD:(h + 1) * HD]
            k = qkvm[:, D + h * HD:D + (h + 1) * HD]
            v = qkvm[:, 2 * D + h * HD:2 * D + (h + 1) * HD]
            att = _dot_t(q, k) / scale
            att = jnp.where(causal, att, -1e9)
            att = jax.nn.softmax(att, axis=-1)
            ys.append(
                jax.lax.dot_general(
                    att.astype(jnp.bfloat16), v.astype(jnp.bfloat16),
                    (((1,), (0,)), ((), ())),
                    preferred_element_type=jnp.float32,
                )
            )
        y = jnp.concatenate(ys, axis=-1)
        x = x + _dot_t(y, proj_ref[0])
        x_s[...] = x
        h_ = _ln(x, ln2g_ref[0, 0], ln2b_ref[0, 0])
        h_s[...] = h_
        scores = _dot_t(h_, rw_ref[0])
        probs = jax.nn.softmax(scores, axis=-1)
        lane = jax.lax.broadcasted_iota(jnp.int32, (T, E), 1)
        big = jnp.int32(2 * E)
        m1 = jnp.max(scores, axis=-1, keepdims=True)
        j1 = jnp.min(jnp.where(scores == m1, lane, big), axis=-1, keepdims=True)
        oh1 = lane == j1
        s2 = jnp.where(oh1, -jnp.inf, scores)
        m2 = jnp.max(s2, axis=-1, keepdims=True)
        j2 = jnp.min(jnp.where(s2 == m2, lane, big), axis=-1, keepdims=True)
        mask = oh1 | (lane == j2)
        mp = jnp.where(mask, probs, 0.0)
        mp = mp / (jnp.sum(mp, axis=-1, keepdims=True) + 1e-8)
        mp_s[...] = mp
        delta_s[...] = jnp.zeros_like(delta_s)

    @pl.when(i < L * S)
    def _experts():
        h_ = h_s[...]
        lane = jax.lax.broadcasted_iota(jnp.int32, (T, E), 1)
        mp = mp_s[...]
        acc = delta_s[...]
        for j in range(EPB):
            a1 = jnp.maximum(_dot_t(h_, fc1_ref[0, j]), 0.0)
            eo = _dot_t(a1, fc2_ref[0, j])
            w = jnp.sum(
                jnp.where(lane == e * EPB + j, mp, 0.0), axis=-1,
                keepdims=True,
            )
            acc = acc + w * eo
        delta_s[...] = acc

        @pl.when(e == S - 1)
        def _finish_layer():
            x_s[...] += delta_s[...]

    @pl.when(i >= L * S)
    def _head():
        out_ref[...] = _dot_t(x_s[...], hw_ref[...])


def _fwd(x0, n0g, n0b, ln1g, ln1b, ln2g, ln2b, qkv_w, proj_w, rw, fc1, fc2,
         head_w):
    lS = L * S

    def lidx(i):
        return jnp.minimum(i // S, L - 1)

    def eidx(i):
        return jnp.where(i < lS, jax.lax.rem(i, S), S - 1)

    def hidx(i):
        return jnp.maximum(i - lS, 0)

    return pl.pallas_call(
        _layers_kernel,
        grid=(lS + NH,),
        in_specs=[
            pl.BlockSpec((T, D), lambda i: (0, 0)),
            pl.BlockSpec((1, D), lambda i: (0, 0)),
            pl.BlockSpec((1, D), lambda i: (0, 0)),
            pl.BlockSpec((1, 1, D), lambda i: (lidx(i), 0, 0)),
            pl.BlockSpec((1, 1, D), lambda i: (lidx(i), 0, 0)),
            pl.BlockSpec((1, 1, D), lambda i: (lidx(i), 0, 0)),
            pl.BlockSpec((1, 1, D), lambda i: (lidx(i), 0, 0)),
            pl.BlockSpec((1, 3 * D, D), lambda i: (lidx(i), 0, 0)),
            pl.BlockSpec((1, D, D), lambda i: (lidx(i), 0, 0)),
            pl.BlockSpec((1, E, D), lambda i: (lidx(i), 0, 0)),
            pl.BlockSpec((1, EPB, DFF, D),
                         lambda i: (lidx(i), eidx(i), 0, 0)),
            pl.BlockSpec((1, EPB, D, DFF),
                         lambda i: (lidx(i), eidx(i), 0, 0)),
            pl.BlockSpec((VCHUNK, D), lambda i: (hidx(i), 0)),
        ],
        out_specs=pl.BlockSpec((T, VCHUNK), lambda i: (0, hidx(i))),
        out_shape=jax.ShapeDtypeStruct((T, V), jnp.float32),
        scratch_shapes=[
            pltpu.VMEM((T, D), jnp.float32),
            pltpu.VMEM((T, D), jnp.float32),
            pltpu.VMEM((T, E), jnp.float32),
            pltpu.VMEM((T, D), jnp.float32),
        ],
        compiler_params=pltpu.CompilerParams(
            dimension_semantics=("arbitrary",),
        ),
    )(x0, n0g, n0b, ln1g, ln1b, ln2g, ln2b, qkv_w, proj_w, rw, fc1, fc2,
      head_w)


@jax.jit
def _run(tokens, wte, wpe, norm0_g, norm0_b, ln1_g, ln1_b, ln2_g, ln2_b,
         attn_qkv_w, attn_proj_w, router_w, expert_fc1, expert_fc2, lm_head_w):
    tok = tokens.reshape(T).astype(jnp.int32)
    x0 = _embed(tok, wte, wpe)
    logits = _fwd(
        x0,
        norm0_g.reshape(1, D), norm0_b.reshape(1, D),
        ln1_g.reshape(L, 1, D), ln1_b.reshape(L, 1, D),
        ln2_g.reshape(L, 1, D), ln2_b.reshape(L, 1, D),
        attn_qkv_w, attn_proj_w, router_w, expert_fc1, expert_fc2,
        lm_head_w,
    )
    return logits.reshape(B, T, V)


def kernel(tokens, wte, wpe, norm0_g, norm0_b, ln1_g, ln1_b, ln2_g, ln2_b,
           attn_qkv_w, attn_proj_w, router_w, expert_fc1, expert_fc2,
           lm_head_w):
    return _run(tokens, wte, wpe, norm0_g, norm0_b, ln1_g, ln1_b, ln2_g,
                ln2_b, attn_qkv_w, attn_proj_w, router_w, expert_fc1,
                expert_fc2, lm_head_w)
